# Initial kernel scaffold; baseline (speedup 1.0000x reference)
import jax
import jax.numpy as jnp
from jax.experimental import pallas as pl


def kernel(labels, features_old, features, outputs_old, outputs, prototypes, num_class, num_old_class, num_new_class, epoch, train_step, len_epoch):
    raise NotImplementedError("write your pallas kernel here")


# speedup vs baseline: 23.1828x; 23.1828x over previous
import functools

import jax
import jax.numpy as jnp
from jax import lax
from jax.experimental import pallas as pl
from jax.experimental.pallas import tpu as pltpu

_TEMP = 0.07
_B, _C, _H, _W = 8, 256, 32, 32
_P = 32
_HW = _H * _W


def _body(nc_ref, lab_ref, feat_ref, proto_ref, out_ref, sums_acc, cnt_acc):
    b = pl.program_id(0)
    lab = lab_ref[0, 0]
    feat = feat_ref[0]

    oh = (lab[None, :] == lax.broadcasted_iota(jnp.int32, (_P, _HW), 0)
          ).astype(jnp.float32)
    s = lax.dot_general(oh, feat, (((1,), (1,)), ((), ())),
                        preferred_element_type=jnp.float32)
    sums_acc[b] = s
    cnt_acc[b] = jnp.sum(oh, axis=1)

    @pl.when(b == _B - 1)
    def _loss():
        nc = nc_ref[0]
        sums = sums_acc[...]
        cnt2 = cnt_acc[...]
        cls1 = lax.broadcasted_iota(jnp.int32, (1, _P), 1)
        valid2 = (cls1 >= 1) & (cls1 <= nc)
        pres2 = (cnt2 > 0.5) & valid2
        presf2 = pres2.astype(jnp.float32)
        np_ = jnp.sum(presf2, axis=0)
        nrm = jnp.sqrt(jnp.sum(sums * sums, axis=2, keepdims=True))
        u = sums / jnp.maximum(nrm, 1e-12)
        pr = proto_ref[...]
        pnrm = jnp.sqrt(jnp.sum(pr * pr, axis=1, keepdims=True))
        pn = pr / jnp.maximum(pnrm, 1e-12)

        a_mat = u.reshape(_B * _P, _C)
        g1 = lax.dot_general(a_mat, a_mat, (((1,), (1,)), ((), ())),
                             preferred_element_type=jnp.float32)
        g2 = lax.dot_general(a_mat, pn, (((1,), (1,)), ((), ())),
                             preferred_element_type=jnp.float32)
        e1 = jnp.exp(g1 * (1.0 / _TEMP)).reshape(_B, _P, _B * _P)
        e2 = jnp.exp(g2 * (1.0 / _TEMP)).reshape(_B, _P, _P)

        inv_cnt = 1.0 / (np_ + 1.0)
        w12d = presf2 * inv_cnt[None, :]
        w1f = jnp.concatenate([w12d[i:i + 1, :] for i in range(_B)],
                              axis=1)
        w2 = valid2[0].astype(jnp.float32) * inv_cnt
        den = (jnp.sum(e1 * w1f[None, :, :], axis=2)
               + jnp.sum(e2 * w2[None, None, :], axis=2))

        q = jnp.sum(u, axis=0)
        nm1 = jnp.sum(u * q[None, :, :], axis=2)
        nm2 = jnp.sum(u * pn[None, :, :], axis=2)

        t = np_[None, :] * jnp.log(den) - (nm1 + nm2 - 1.0) * (1.0 / _TEMP)
        cls_sum = jnp.sum(t * presf2, axis=0)
        contrib = cls_sum / jnp.maximum(np_ * np_, 1.0)
        exist = (np_ >= 0.5).astype(jnp.float32)
        loss = 0.1 * jnp.sum(contrib * exist) / jnp.sum(exist)
        out_ref[0, 0] = loss


@functools.partial(jax.jit, static_argnames=())
def _run(labels, features, prototypes, num_class):
    lab_ds = labels.reshape(_B, _H, 16, _W, 16)[:, :, 0, :, 0]
    lab_ds = lab_ds.reshape(_B, 1, _HW).astype(jnp.int32)
    feat = features.reshape(_B, _C, _HW)
    proto_p = jnp.zeros((_P, _C), jnp.float32).at[:_P - 11].set(
        prototypes.astype(jnp.float32))
    nc_arr = jnp.asarray(num_class, jnp.int32).reshape(1)

    out = pl.pallas_call(
        _body,
        grid=(_B,),
        in_specs=[
            pl.BlockSpec(memory_space=pltpu.SMEM),
            pl.BlockSpec((1, 1, _HW), lambda b: (b, 0, 0)),
            pl.BlockSpec((1, _C, _HW), lambda b: (b, 0, 0)),
            pl.BlockSpec((_P, _C), lambda b: (0, 0)),
        ],
        out_specs=pl.BlockSpec(memory_space=pltpu.SMEM),
        out_shape=jax.ShapeDtypeStruct((1, 1), jnp.float32),
        scratch_shapes=[
            pltpu.VMEM((_B, _P, _C), jnp.float32),
            pltpu.VMEM((_B, _P), jnp.float32),
        ],
    )(nc_arr, lab_ds, feat, proto_p)
    return out[0, 0]


def kernel(labels, features_old, features, outputs_old, outputs, prototypes,
           num_class, num_old_class, num_new_class, epoch, train_step,
           len_epoch):
    return _run(labels, features, prototypes, num_class)


# speedup vs baseline: 23.8209x; 1.0275x over previous
import functools

import jax
import jax.numpy as jnp
from jax import lax
from jax.experimental import pallas as pl
from jax.experimental.pallas import tpu as pltpu

_TEMP = 0.07
_B, _C, _H, _W = 8, 256, 32, 32
_P = 32
_HW = _H * _W


def _body(nc_ref, lab_ref, feat_ref, proto_ref, out_ref):
    lab = lab_ref[:, 0, :]
    feat = feat_ref[...]

    oh = (lab[:, None, :] == lax.broadcasted_iota(jnp.int32, (_B, _P, _HW), 1)
          ).astype(jnp.float32)
    sums = lax.dot_general(oh, feat, (((2,), (2,)), ((0,), (0,))),
                           preferred_element_type=jnp.float32)
    cnt2 = jnp.sum(oh, axis=2)

    nc = nc_ref[0]
    cls1 = lax.broadcasted_iota(jnp.int32, (1, _P), 1)
    valid2 = (cls1 >= 1) & (cls1 <= nc)
    pres2 = (cnt2 > 0.5) & valid2
    presf2 = pres2.astype(jnp.float32)
    np_ = jnp.sum(presf2, axis=0)
    nrm = jnp.sqrt(jnp.sum(sums * sums, axis=2, keepdims=True))
    u = sums / jnp.maximum(nrm, 1e-12)
    pr = proto_ref[...]
    pnrm = jnp.sqrt(jnp.sum(pr * pr, axis=1, keepdims=True))
    pn = pr / jnp.maximum(pnrm, 1e-12)

    a_mat = u.reshape(_B * _P, _C)
    g1 = lax.dot_general(a_mat, a_mat, (((1,), (1,)), ((), ())),
                         preferred_element_type=jnp.float32)
    g2 = lax.dot_general(a_mat, pn, (((1,), (1,)), ((), ())),
                         preferred_element_type=jnp.float32)
    e1 = jnp.exp(g1 * (1.0 / _TEMP)).reshape(_B, _P, _B * _P)
    e2 = jnp.exp(g2 * (1.0 / _TEMP)).reshape(_B, _P, _P)

    inv_cnt = 1.0 / (np_ + 1.0)
    w12d = presf2 * inv_cnt[None, :]
    w1f = jnp.concatenate([w12d[i:i + 1, :] for i in range(_B)],
                          axis=1)
    w2 = valid2[0].astype(jnp.float32) * inv_cnt
    den = (jnp.sum(e1 * w1f[None, :, :], axis=2)
           + jnp.sum(e2 * w2[None, None, :], axis=2))

    q = jnp.sum(u, axis=0)
    nm1 = jnp.sum(u * q[None, :, :], axis=2)
    nm2 = jnp.sum(u * pn[None, :, :], axis=2)

    t = np_[None, :] * jnp.log(den) - (nm1 + nm2 - 1.0) * (1.0 / _TEMP)
    cls_sum = jnp.sum(t * presf2, axis=0)
    contrib = cls_sum / jnp.maximum(np_ * np_, 1.0)
    exist = (np_ >= 0.5).astype(jnp.float32)
    loss = 0.1 * jnp.sum(contrib * exist) / jnp.sum(exist)
    out_ref[0, 0] = loss


@jax.jit
def _run(labels, features, prototypes, num_class):
    lab_ds = labels.reshape(_B, _H, 16, _W, 16)[:, :, 0, :, 0]
    lab_ds = lab_ds.reshape(_B, 1, _HW).astype(jnp.int32)
    feat = features.reshape(_B, _C, _HW)
    proto_p = jnp.zeros((_P, _C), jnp.float32).at[:21].set(
        prototypes.astype(jnp.float32))
    nc_arr = jnp.asarray(num_class, jnp.int32).reshape(1)

    out = pl.pallas_call(
        _body,
        in_specs=[
            pl.BlockSpec(memory_space=pltpu.SMEM),
            pl.BlockSpec((_B, 1, _HW), lambda: (0, 0, 0)),
            pl.BlockSpec((_B, _C, _HW), lambda: (0, 0, 0)),
            pl.BlockSpec((_P, _C), lambda: (0, 0)),
        ],
        out_specs=pl.BlockSpec(memory_space=pltpu.SMEM),
        out_shape=jax.ShapeDtypeStruct((1, 1), jnp.float32),
    )(nc_arr, lab_ds, feat, proto_p)
    return out[0, 0]


def kernel(labels, features_old, features, outputs_old, outputs, prototypes,
           num_class, num_old_class, num_new_class, epoch, train_step,
           len_epoch):
    return _run(labels, features, prototypes, num_class)


# speedup vs baseline: 66.1046x; 2.7751x over previous
import functools

import jax
import jax.numpy as jnp
from jax import lax
from jax.experimental import pallas as pl
from jax.experimental.pallas import tpu as pltpu

_TEMP = 0.07
_B, _C, _H, _W = 8, 256, 32, 32
_P = 32
_HW = _H * _W


def _body(nc_ref, lab_ref, feat_ref, proto_ref, out_ref):
    lab = lab_ref[:, 0, :]
    feat = feat_ref[...]

    oh = (lab[:, None, :] == lax.broadcasted_iota(jnp.int32, (_B, _P, _HW), 1)
          ).astype(jnp.float32)
    sums = lax.dot_general(oh, feat, (((2,), (2,)), ((0,), (0,))),
                           preferred_element_type=jnp.float32)
    cnt2 = jnp.sum(oh, axis=2)

    nc = nc_ref[0]
    cls1 = lax.broadcasted_iota(jnp.int32, (1, _P), 1)
    valid2 = (cls1 >= 1) & (cls1 <= nc)
    pres2 = (cnt2 > 0.5) & valid2
    presf2 = pres2.astype(jnp.float32)
    np_ = jnp.sum(presf2, axis=0)
    nrm = jnp.sqrt(jnp.sum(sums * sums, axis=2, keepdims=True))
    u = sums / jnp.maximum(nrm, 1e-12)
    pr = proto_ref[...]
    pnrm = jnp.sqrt(jnp.sum(pr * pr, axis=1, keepdims=True))
    pn = pr / jnp.maximum(pnrm, 1e-12)

    a_mat = u.reshape(_B * _P, _C)
    g1 = lax.dot_general(a_mat, a_mat, (((1,), (1,)), ((), ())),
                         preferred_element_type=jnp.float32)
    g2 = lax.dot_general(a_mat, pn, (((1,), (1,)), ((), ())),
                         preferred_element_type=jnp.float32)
    e1 = jnp.exp(g1 * (1.0 / _TEMP)).reshape(_B, _P, _B * _P)
    e2 = jnp.exp(g2 * (1.0 / _TEMP)).reshape(_B, _P, _P)

    inv_cnt = 1.0 / (np_ + 1.0)
    w12d = presf2 * inv_cnt[None, :]
    w1f = jnp.concatenate([w12d[i:i + 1, :] for i in range(_B)],
                          axis=1)
    w2 = valid2[0].astype(jnp.float32) * inv_cnt
    den = (jnp.sum(e1 * w1f[None, :, :], axis=2)
           + jnp.sum(e2 * w2[None, None, :], axis=2))

    q = jnp.sum(u, axis=0)
    nm1 = jnp.sum(u * q[None, :, :], axis=2)
    nm2 = jnp.sum(u * pn[None, :, :], axis=2)

    t = np_[None, :] * jnp.log(den) - (nm1 + nm2 - 1.0) * (1.0 / _TEMP)
    cls_sum = jnp.sum(t * presf2, axis=0)
    contrib = cls_sum / jnp.maximum(np_ * np_, 1.0)
    exist = (np_ >= 0.5).astype(jnp.float32)
    loss = 0.1 * jnp.sum(contrib * exist) / jnp.sum(exist)
    out_ref[0, 0] = loss


@jax.jit
def _run(labels, features, prototypes, num_class):
    lab_rows = lax.optimization_barrier(labels[:, ::16, :])
    lab_ds = lab_rows[:, :, ::16]
    lab_ds = lab_ds.reshape(_B, 1, _HW).astype(jnp.int32)
    feat = features.reshape(_B, _C, _HW)
    proto_p = jnp.zeros((_P, _C), jnp.float32).at[:21].set(
        prototypes.astype(jnp.float32))
    nc_arr = jnp.asarray(num_class, jnp.int32).reshape(1)

    out = pl.pallas_call(
        _body,
        in_specs=[
            pl.BlockSpec(memory_space=pltpu.SMEM),
            pl.BlockSpec((_B, 1, _HW), lambda: (0, 0, 0)),
            pl.BlockSpec((_B, _C, _HW), lambda: (0, 0, 0)),
            pl.BlockSpec((_P, _C), lambda: (0, 0)),
        ],
        out_specs=pl.BlockSpec(memory_space=pltpu.SMEM),
        out_shape=jax.ShapeDtypeStruct((1, 1), jnp.float32),
    )(nc_arr, lab_ds, feat, proto_p)
    return out[0, 0]


def kernel(labels, features_old, features, outputs_old, outputs, prototypes,
           num_class, num_old_class, num_new_class, epoch, train_step,
           len_epoch):
    return _run(labels, features, prototypes, num_class)
